# baseline (speedup 1.0000x reference)
import jax
import jax.numpy as jnp
from jax import lax
from jax.experimental import pallas as pl
from jax.experimental.pallas import tpu as pltpu
from jax.experimental.pallas import tpu_sc as plsc

N = 100000
D = 128
NUM_CLASSES = 5
EMB_DIM = 125
AW = 26
ROWS_W = N // 25
CHUNK = 400
ROUNDS = ROWS_W // CHUNK
TILES = N // CHUNK


def _body(pt_hbm, eta_hbm, phi_hbm, cls_hbm, tab_hbm, out_hbm,
          idx_v, pt_v, eta_v, phi_v, rows_v, tab125_v, tab_tmp, tab_v,
          sem_g, sem_out):
    sid = lax.axis_index("s")
    wid = sid * 2 + lax.axis_index("c")

    @pl.when(sid == 0)
    def _():
        pltpu.sync_copy(tab_hbm, tab125_v)
        zeros16 = jnp.zeros((16,), jnp.float32)
        for row in range(NUM_CLASSES):
            tab_tmp[row, pl.ds(0, 16)] = zeros16
            for j in range(8):
                src0 = min(16 * j, EMB_DIM - 16)
                tab_tmp[row, pl.ds(3 + src0, 16)] = tab125_v[row, pl.ds(src0, 16)]
        pltpu.sync_copy(tab_tmp, tab_v)

    plsc.subcore_barrier()

    lanes = lax.iota(jnp.int32, 16)
    col0 = jnp.zeros((16,), jnp.int32)
    col1 = jnp.full((16,), 1, jnp.int32)
    col2 = jnp.full((16,), 2, jnp.int32)

    def pipeline(base, tile0, rounds):
        nrows = rounds * CHUNK
        pltpu.sync_copy(cls_hbm.at[pl.ds(base, nrows)], idx_v.at[pl.ds(0, nrows)])
        pltpu.sync_copy(pt_hbm.at[pl.ds(base, nrows)], pt_v.at[pl.ds(0, nrows)])
        pltpu.sync_copy(eta_hbm.at[pl.ds(base, nrows)], eta_v.at[pl.ds(0, nrows)])
        pltpu.sync_copy(phi_hbm.at[pl.ds(base, nrows)], phi_v.at[pl.ds(0, nrows)])

        def start_gather(r):
            b = r & 1
            return pltpu.async_copy(
                tab_v.at[idx_v.at[pl.ds(r * CHUNK, CHUNK)]], rows_v[b], sem_g[b])

        def wait_out(b):
            pltpu.make_async_copy(rows_v[b], out_hbm.at[0], sem_out[b]).wait()

        gathers = [None] * rounds
        gathers[0] = start_gather(0)
        for r in range(rounds):
            b = r & 1
            gathers[r].wait()
            if r + 1 < rounds:
                if r >= 1:
                    wait_out(1 - b)
                gathers[r + 1] = start_gather(r + 1)
            off = r * CHUNK
            for j in range(CHUNK // 16):
                rows16 = lanes + (j * 16)
                plsc.store_scatter(rows_v[b], [rows16, col0], pt_v[pl.ds(off + j * 16, 16)])
                plsc.store_scatter(rows_v[b], [rows16, col1], eta_v[pl.ds(off + j * 16, 16)])
                plsc.store_scatter(rows_v[b], [rows16, col2], phi_v[pl.ds(off + j * 16, 16)])
            pltpu.async_copy(rows_v[b], out_hbm.at[tile0 + r], sem_out[b])

        wait_out(1 - (rounds & 1))
        wait_out(rounds & 1)

    @pl.when(wid < AW - 2)
    def _():
        pipeline(wid * ROWS_W, wid * ROUNDS, ROUNDS)

    @pl.when((wid == AW - 2) | (wid == AW - 1))
    def _():
        half = ROWS_W // 2
        k = wid - (AW - 2)
        pipeline((AW - 2) * ROWS_W + k * half,
                 (AW - 2) * ROUNDS + k * (ROUNDS // 2), ROUNDS // 2)


@jax.jit
def kernel(pt, eta, phi, cls, class_embedding):
    mesh = plsc.VectorSubcoreMesh(core_axis_name="c", subcore_axis_name="s",
                                  num_cores=2, num_subcores=16)
    run = pl.kernel(
        _body,
        out_type=jax.ShapeDtypeStruct((TILES, CHUNK, D), jnp.float32),
        mesh=mesh,
        scratch_types=[
            pltpu.VMEM((ROWS_W,), jnp.int32),
            pltpu.VMEM((ROWS_W,), jnp.float32),
            pltpu.VMEM((ROWS_W,), jnp.float32),
            pltpu.VMEM((ROWS_W,), jnp.float32),
            [pltpu.VMEM((CHUNK, D), jnp.float32)] * 2,
            pltpu.VMEM((NUM_CLASSES, EMB_DIM), jnp.float32),
            pltpu.VMEM((NUM_CLASSES, D), jnp.float32),
            pltpu.VMEM_SHARED((NUM_CLASSES, D), jnp.float32),
            [pltpu.SemaphoreType.DMA] * 2,
            [pltpu.SemaphoreType.DMA] * 2,
        ],
        compiler_params=pltpu.CompilerParams(needs_layout_passes=False),
    )
    return run(pt, eta, phi, cls, class_embedding).reshape(N, D)


# speedup vs baseline: 1.1101x; 1.1101x over previous
import jax
import jax.numpy as jnp
from jax import lax
from jax.experimental import pallas as pl
from jax.experimental.pallas import tpu as pltpu
from jax.experimental.pallas import tpu_sc as plsc

N = 100000
D = 128
NUM_CLASSES = 5
EMB_DIM = 125
AW = 25
ROWS_W = N // 25
CHUNK = 400
ROUNDS = ROWS_W // CHUNK
TILES = N // CHUNK


def _body(pt_hbm, eta_hbm, phi_hbm, cls_hbm, tab_hbm, out_hbm,
          idx_v, pt_v, eta_v, phi_v, rows_v, tab125_v, tab_tmp, tab_v,
          sem_g, sem_out):
    sid = lax.axis_index("s")
    wid = sid * 2 + lax.axis_index("c")

    @pl.when(sid == 0)
    def _():
        pltpu.sync_copy(tab_hbm, tab125_v)
        zeros16 = jnp.zeros((16,), jnp.float32)
        for row in range(NUM_CLASSES):
            tab_tmp[row, pl.ds(0, 16)] = zeros16
            for j in range(8):
                src0 = min(16 * j, EMB_DIM - 16)
                tab_tmp[row, pl.ds(3 + src0, 16)] = tab125_v[row, pl.ds(src0, 16)]
        pltpu.sync_copy(tab_tmp, tab_v)

    plsc.subcore_barrier()

    lanes = lax.iota(jnp.int32, 16)
    col0 = jnp.zeros((16,), jnp.int32)
    col1 = jnp.full((16,), 1, jnp.int32)
    col2 = jnp.full((16,), 2, jnp.int32)

    def pipeline(base, tile0, rounds):
        nrows = rounds * CHUNK
        pltpu.sync_copy(cls_hbm.at[pl.ds(base, nrows)], idx_v.at[pl.ds(0, nrows)])
        pltpu.sync_copy(pt_hbm.at[pl.ds(base, nrows)], pt_v.at[pl.ds(0, nrows)])
        pltpu.sync_copy(eta_hbm.at[pl.ds(base, nrows)], eta_v.at[pl.ds(0, nrows)])
        pltpu.sync_copy(phi_hbm.at[pl.ds(base, nrows)], phi_v.at[pl.ds(0, nrows)])

        def start_gather(r):
            b = r & 1
            return pltpu.async_copy(
                tab_v.at[idx_v.at[pl.ds(r * CHUNK, CHUNK)]], rows_v[b], sem_g[b])

        def wait_out(b):
            pltpu.make_async_copy(rows_v[b], out_hbm.at[0], sem_out[b]).wait()

        gathers = [None] * rounds
        gathers[0] = start_gather(0)
        for r in range(rounds):
            b = r & 1
            gathers[r].wait()
            if r + 1 < rounds:
                if r >= 1:
                    wait_out(1 - b)
                gathers[r + 1] = start_gather(r + 1)
            off = r * CHUNK

            def fix(j, carry, b=b):
                rows16 = lanes + j * 16
                src = off + j * 16
                plsc.store_scatter(rows_v[b], [rows16, col0], pt_v[pl.ds(src, 16)])
                plsc.store_scatter(rows_v[b], [rows16, col1], eta_v[pl.ds(src, 16)])
                plsc.store_scatter(rows_v[b], [rows16, col2], phi_v[pl.ds(src, 16)])
                return carry

            lax.fori_loop(0, CHUNK // 16, fix, 0)
            pltpu.async_copy(rows_v[b], out_hbm.at[tile0 + r], sem_out[b])

        wait_out(1 - (rounds & 1))
        wait_out(rounds & 1)

    @pl.when(wid < AW)
    def _():
        pipeline(wid * ROWS_W, wid * ROUNDS, ROUNDS)


@jax.jit
def kernel(pt, eta, phi, cls, class_embedding):
    mesh = plsc.VectorSubcoreMesh(core_axis_name="c", subcore_axis_name="s",
                                  num_cores=2, num_subcores=16)
    run = pl.kernel(
        _body,
        out_type=jax.ShapeDtypeStruct((TILES, CHUNK, D), jnp.float32),
        mesh=mesh,
        scratch_types=[
            pltpu.VMEM((ROWS_W,), jnp.int32),
            pltpu.VMEM((ROWS_W,), jnp.float32),
            pltpu.VMEM((ROWS_W,), jnp.float32),
            pltpu.VMEM((ROWS_W,), jnp.float32),
            [pltpu.VMEM((CHUNK, D), jnp.float32)] * 2,
            pltpu.VMEM((NUM_CLASSES, EMB_DIM), jnp.float32),
            pltpu.VMEM((NUM_CLASSES, D), jnp.float32),
            pltpu.VMEM_SHARED((NUM_CLASSES, D), jnp.float32),
            [pltpu.SemaphoreType.DMA] * 2,
            [pltpu.SemaphoreType.DMA] * 2,
        ],
        compiler_params=pltpu.CompilerParams(needs_layout_passes=False),
    )
    return run(pt, eta, phi, cls, class_embedding).reshape(N, D)


# speedup vs baseline: 1.1928x; 1.0745x over previous
import jax
import jax.numpy as jnp
from jax import lax
from jax.experimental import pallas as pl
from jax.experimental.pallas import tpu as pltpu
from jax.experimental.pallas import tpu_sc as plsc

N = 100000
D = 128
NUM_CLASSES = 5
EMB_DIM = 125
CHUNK = 400
TILES = N // CHUNK
HI_W = 26
HI_ROUNDS = 8
LO_ROUNDS = 7
ROWS_MAX = HI_ROUNDS * CHUNK


def _body(pt_hbm, eta_hbm, phi_hbm, cls_hbm, tab_hbm, out_hbm,
          idx_v, pt_v, eta_v, phi_v, rows_v, tab125_v, tab_tmp, tab_v,
          sem_g, sem_out):
    sid = lax.axis_index("s")
    wid = sid * 2 + lax.axis_index("c")

    @pl.when(sid == 0)
    def _():
        pltpu.sync_copy(tab_hbm, tab125_v)
        zeros16 = jnp.zeros((16,), jnp.float32)
        for row in range(NUM_CLASSES):
            tab_tmp[row, pl.ds(0, 16)] = zeros16
            for j in range(8):
                src0 = min(16 * j, EMB_DIM - 16)
                tab_tmp[row, pl.ds(3 + src0, 16)] = tab125_v[row, pl.ds(src0, 16)]
        pltpu.sync_copy(tab_tmp, tab_v)

    plsc.subcore_barrier()

    lanes = lax.iota(jnp.int32, 16)
    col0 = jnp.zeros((16,), jnp.int32)
    col1 = jnp.full((16,), 1, jnp.int32)
    col2 = jnp.full((16,), 2, jnp.int32)

    def pipeline(base, tile0, rounds):
        nrows = rounds * CHUNK
        pltpu.sync_copy(cls_hbm.at[pl.ds(base, nrows)], idx_v.at[pl.ds(0, nrows)])
        pltpu.sync_copy(pt_hbm.at[pl.ds(base, nrows)], pt_v.at[pl.ds(0, nrows)])
        pltpu.sync_copy(eta_hbm.at[pl.ds(base, nrows)], eta_v.at[pl.ds(0, nrows)])
        pltpu.sync_copy(phi_hbm.at[pl.ds(base, nrows)], phi_v.at[pl.ds(0, nrows)])

        def start_gather(r):
            b = r & 1
            return pltpu.async_copy(
                tab_v.at[idx_v.at[pl.ds(r * CHUNK, CHUNK)]], rows_v[b], sem_g[b])

        def wait_out(b):
            pltpu.make_async_copy(rows_v[b], out_hbm.at[0], sem_out[b]).wait()

        gathers = [None] * rounds
        gathers[0] = start_gather(0)
        for r in range(rounds):
            b = r & 1
            gathers[r].wait()
            if r + 1 < rounds:
                if r >= 1:
                    wait_out(1 - b)
                gathers[r + 1] = start_gather(r + 1)
            off = r * CHUNK

            def fix(j, carry, b=b):
                rows16 = lanes + j * 16
                src = off + j * 16
                plsc.store_scatter(rows_v[b], [rows16, col0], pt_v[pl.ds(src, 16)])
                plsc.store_scatter(rows_v[b], [rows16, col1], eta_v[pl.ds(src, 16)])
                plsc.store_scatter(rows_v[b], [rows16, col2], phi_v[pl.ds(src, 16)])
                return carry

            lax.fori_loop(0, CHUNK // 16, fix, 0)
            pltpu.async_copy(rows_v[b], out_hbm.at[tile0 + r], sem_out[b])

        wait_out(1 - (rounds & 1))
        wait_out(rounds & 1)

    @pl.when(wid < HI_W)
    def _():
        pipeline(wid * HI_ROUNDS * CHUNK, wid * HI_ROUNDS, HI_ROUNDS)

    @pl.when(wid >= HI_W)
    def _():
        tile0 = HI_W * HI_ROUNDS + (wid - HI_W) * LO_ROUNDS
        pipeline(tile0 * CHUNK, tile0, LO_ROUNDS)


@jax.jit
def kernel(pt, eta, phi, cls, class_embedding):
    mesh = plsc.VectorSubcoreMesh(core_axis_name="c", subcore_axis_name="s",
                                  num_cores=2, num_subcores=16)
    run = pl.kernel(
        _body,
        out_type=jax.ShapeDtypeStruct((TILES, CHUNK, D), jnp.float32),
        mesh=mesh,
        scratch_types=[
            pltpu.VMEM((ROWS_MAX,), jnp.int32),
            pltpu.VMEM((ROWS_MAX,), jnp.float32),
            pltpu.VMEM((ROWS_MAX,), jnp.float32),
            pltpu.VMEM((ROWS_MAX,), jnp.float32),
            [pltpu.VMEM((CHUNK, D), jnp.float32)] * 2,
            pltpu.VMEM((NUM_CLASSES, EMB_DIM), jnp.float32),
            pltpu.VMEM((NUM_CLASSES, D), jnp.float32),
            pltpu.VMEM_SHARED((NUM_CLASSES, D), jnp.float32),
            [pltpu.SemaphoreType.DMA] * 2,
            [pltpu.SemaphoreType.DMA] * 2,
        ],
        compiler_params=pltpu.CompilerParams(needs_layout_passes=False),
    )
    return run(pt, eta, phi, cls, class_embedding).reshape(N, D)
